# baseline (speedup 1.0000x reference)
import functools

import jax
import jax.numpy as jnp
from jax import lax
from jax.experimental import pallas as pl
from jax.experimental.pallas import tpu as pltpu
from jax.experimental.pallas import tpu_sc as plsc

N = 10000
E = 160000
NP = 10240
EP = 163840
IN_DIM = 256
MID_DIM = 256
OUT_DIM = 128
PROJ_DIM = 256

NC = 2
NS = 16
CHUNK = 128
NBUF = 2
DQ = 64
ROWS_PER_SUB = NP // NS
PAD_ROW = NP - 1

_MESH = plsc.VectorSubcoreMesh(
    core_axis_name="c", subcore_axis_name="s", num_cores=NC, num_subcores=NS)
_SC_PARAMS = pltpu.CompilerParams(use_tc_tiling_on_sc=False)

R = 2048
_F32 = jnp.float32


def _deg_body(dst_hbm, zrow_hbm, out_hbm, acc, zbuf, ones_v, dst_v):
    c = lax.axis_index("c")
    s = lax.axis_index("s")
    pltpu.sync_copy(zrow_hbm.at[0],
                    acc.at[pl.ds(s * ROWS_PER_SUB, ROWS_PER_SUB)])
    for j in range(CHUNK // 16):
        ones_v[pl.ds(j * 16, 16)] = jnp.ones((16,), _F32)
    plsc.subcore_barrier()

    nchunks = EP // (NC * NS * CHUNK)
    base = (c * NS + s) * nchunks
    pltpu.sync_copy(dst_hbm.at[pl.ds(base, nchunks)], dst_v)

    def body(i, carry):
        pltpu.sync_copy(ones_v, acc.at[dst_v.at[i]], add=True)
        return carry

    lax.fori_loop(0, nchunks, body, 0)
    plsc.subcore_barrier()
    row0 = s * ROWS_PER_SUB
    pltpu.sync_copy(acc.at[pl.ds(row0, ROWS_PER_SUB)], zbuf)
    pltpu.sync_copy(zbuf, out_hbm.at[pl.ds(c * NP + row0, ROWS_PER_SUB)])


def _deg_kernel(dst2d, zrow1):
    return pl.kernel(
        _deg_body,
        out_type=jax.ShapeDtypeStruct((NC * NP,), _F32),
        mesh=_MESH,
        scratch_types=[
            pltpu.VMEM_SHARED((NP,), _F32),
            pltpu.VMEM((ROWS_PER_SUB,), _F32),
            pltpu.VMEM((CHUNK,), _F32),
            pltpu.VMEM((EP // (NC * NS * CHUNK), CHUNK), jnp.int32),
        ],
    )(dst2d, zrow1)


def _scatter_body(nphases,
                  table_hbm, idx_hbm, dst_hbm, zrows_hbm, out_hbm,
                  table_s, acc, idx_v, dst_v, *bufs_and_sems):
    c = lax.axis_index("c")
    s = lax.axis_index("s")
    n = EP // NS // CHUNK
    base = s * n
    row0 = s * ROWS_PER_SUB

    rows = bufs_and_sems[0:NBUF]
    gsem = bufs_and_sems[NBUF:2 * NBUF]
    isem = bufs_and_sems[2 * NBUF:3 * NBUF]
    dsem = bufs_and_sems[3 * NBUF:4 * NBUF]
    ssem = bufs_and_sems[4 * NBUF:5 * NBUF]

    def idx_load(i, p):
        pltpu.async_copy(idx_hbm.at[pl.ds(base + i, 1)],
                         idx_v.at[pl.ds(p, 1)], isem[p])

    def dst_load(i, p):
        pltpu.async_copy(dst_hbm.at[pl.ds(base + i, 1)],
                         dst_v.at[pl.ds(p, 1)], dsem[p])

    def idx_wait(p):
        pltpu.make_async_copy(idx_hbm.at[pl.ds(base, 1)],
                              idx_v.at[pl.ds(p, 1)], isem[p]).wait()

    def dst_wait(p):
        pltpu.make_async_copy(dst_hbm.at[pl.ds(base, 1)],
                              dst_v.at[pl.ds(p, 1)], dsem[p]).wait()

    def gather(p):
        pltpu.async_copy(table_s.at[idx_v.at[p]], rows[p], gsem[p])

    def gwait(p):
        pltpu.make_async_copy(table_s.at[idx_v.at[p]], rows[p],
                              gsem[p]).wait()

    def scatter(p):
        pltpu.async_copy(rows[p], acc.at[dst_v.at[p]], ssem[p], add=True)

    def swait(p):
        pltpu.make_async_copy(rows[p], acc.at[dst_v.at[p]], ssem[p]).wait()

    for phase in range(nphases):
        q = nphases * c + phase
        pltpu.sync_copy(table_hbm.at[pl.ds(q * NP + row0, ROWS_PER_SUB)],
                        table_s.at[pl.ds(row0, ROWS_PER_SUB)])

        def zinit(j, carry):
            pltpu.sync_copy(zrows_hbm,
                            acc.at[pl.ds(row0 + j * CHUNK, CHUNK)])
            return carry

        lax.fori_loop(0, ROWS_PER_SUB // CHUNK, zinit, 0)
        plsc.subcore_barrier()

        pltpu.sync_copy(idx_hbm.at[pl.ds(base, NBUF)], idx_v)
        for p in range(NBUF):
            gather(p)
            dst_load(p, p)

        def half(i, p):
            gwait(p)

            @pl.when(i + NBUF < n)
            def _():
                idx_load(i + NBUF, p)

            dst_wait(p)
            scatter(p)

        def refill(i, p):
            @pl.when(i + NBUF < n)
            def _():
                swait(p)
                dst_load(i + NBUF, p)
                idx_wait(p)
                gather(p)

        def body(k, carry):
            i0 = NBUF * k
            for p in range(NBUF):
                half(i0 + p, p)
            for p in range(NBUF):
                refill(i0 + p, p)
            return carry

        lax.fori_loop(0, n // NBUF, body, 0)
        for p in range(NBUF):
            swait(p)
        plsc.subcore_barrier()

        def wb(j, carry):
            pltpu.sync_copy(acc.at[pl.ds(row0 + j * CHUNK, CHUNK)], rows[0])
            pltpu.sync_copy(
                rows[0],
                out_hbm.at[pl.ds(q * NP + row0 + j * CHUNK, CHUNK)])
            return carry

        lax.fori_loop(0, ROWS_PER_SUB // CHUNK, wb, 0)
        if phase + 1 < nphases:
            plsc.subcore_barrier()


def _scatter_kernel(table, idx2d, dst2d, zrows, nphases):
    body = functools.partial(_scatter_body, nphases)
    nq = nphases * NC
    return pl.kernel(
        body,
        out_type=jax.ShapeDtypeStruct((nq * NP, DQ), _F32),
        mesh=_MESH,
        compiler_params=_SC_PARAMS,
        scratch_types=(
            [
                pltpu.VMEM_SHARED((NP, DQ), _F32),
                pltpu.VMEM_SHARED((NP, DQ), _F32),
                pltpu.VMEM((NBUF, CHUNK), jnp.int32),
                pltpu.VMEM((NBUF, CHUNK), jnp.int32),
            ]
            + [pltpu.VMEM((CHUNK, DQ), _F32) for _ in range(NBUF)]
            + [pltpu.SemaphoreType.DMA for _ in range(4 * NBUF)]
        ),
    )(table, idx2d, dst2d, zrows)


def _dinv_of(deg2_ref):
    deg = deg2_ref[0, :] + deg2_ref[1, :] + 1.0
    return lax.rsqrt(deg)


def _tc1_body(x_ref, w1_ref, deg2_ref, wf1_ref, wf2_ref, bf1_ref, bf2_ref,
              g1_ref, wc_ref, bc_ref):
    dinv = _dinv_of(deg2_ref)
    h = jnp.dot(x_ref[...], w1_ref[...], preferred_element_type=_F32)
    g = h * dinv[:, None]
    for q in range(4):
        g1_ref[q] = g[:, q * DQ:(q + 1) * DQ]

    @pl.when(pl.program_id(0) == 0)
    def _():
        wc_ref[...] = jnp.dot(wf1_ref[...], wf2_ref[...],
                              preferred_element_type=_F32)
        bc_ref[...] = jnp.dot(bf1_ref[...], wf2_ref[...],
                              preferred_element_type=_F32) + bf2_ref[...]


def _tc1(x_pad, W1, deg2, Wf1, Wf2, bf1r, bf2r):
    return pl.pallas_call(
        _tc1_body,
        grid=(NP // R,),
        in_specs=[
            pl.BlockSpec((R, IN_DIM), lambda i: (i, 0)),
            pl.BlockSpec((IN_DIM, MID_DIM), lambda i: (0, 0)),
            pl.BlockSpec((NC, R), lambda i: (0, i)),
            pl.BlockSpec((OUT_DIM, 256), lambda i: (0, 0)),
            pl.BlockSpec((256, PROJ_DIM), lambda i: (0, 0)),
            pl.BlockSpec((1, 256), lambda i: (0, 0)),
            pl.BlockSpec((1, PROJ_DIM), lambda i: (0, 0)),
        ],
        out_specs=[
            pl.BlockSpec((4, R, DQ), lambda i: (0, i, 0)),
            pl.BlockSpec((OUT_DIM, PROJ_DIM), lambda i: (0, 0)),
            pl.BlockSpec((1, PROJ_DIM), lambda i: (0, 0)),
        ],
        out_shape=[
            jax.ShapeDtypeStruct((4, NP, DQ), _F32),
            jax.ShapeDtypeStruct((OUT_DIM, PROJ_DIM), _F32),
            jax.ShapeDtypeStruct((1, PROJ_DIM), _F32),
        ],
    )(x_pad, W1, deg2, Wf1, Wf2, bf1r, bf2r)


def _tc2_body(s1_ref, g1_ref, deg2_ref, b1_ref, w2_ref, g2_ref):
    dinv = _dinv_of(deg2_ref)
    h = None
    for q in range(4):
        t = jnp.maximum((s1_ref[q] + g1_ref[q]) * dinv[:, None]
                        + b1_ref[0:1, q * DQ:(q + 1) * DQ], 0.0)
        part = jnp.dot(t, w2_ref[q * DQ:(q + 1) * DQ, :],
                       preferred_element_type=_F32)
        h = part if h is None else h + part
    g2 = h * dinv[:, None]
    g2_ref[0] = g2[:, :DQ]
    g2_ref[1] = g2[:, DQ:]


def _tc2(S1, g1, deg2, b1r, W2):
    return pl.pallas_call(
        _tc2_body,
        grid=(NP // R,),
        in_specs=[
            pl.BlockSpec((4, R, DQ), lambda i: (0, i, 0)),
            pl.BlockSpec((4, R, DQ), lambda i: (0, i, 0)),
            pl.BlockSpec((NC, R), lambda i: (0, i)),
            pl.BlockSpec((1, MID_DIM), lambda i: (0, 0)),
            pl.BlockSpec((MID_DIM, OUT_DIM), lambda i: (0, 0)),
        ],
        out_specs=pl.BlockSpec((2, R, DQ), lambda i: (0, i, 0)),
        out_shape=jax.ShapeDtypeStruct((2, NP, DQ), _F32),
    )(S1, g1, deg2, b1r, W2)


def _tc3_body(s2_ref, g2_ref, deg2_ref, b2_ref, wc_ref, bc_ref, out_ref):
    dinv = _dinv_of(deg2_ref)
    acc = None
    for hh in range(2):
        t = jnp.maximum((s2_ref[hh] + g2_ref[hh]) * dinv[:, None]
                        + b2_ref[0:1, hh * DQ:(hh + 1) * DQ], 0.0)
        part = jnp.dot(t, wc_ref[hh * DQ:(hh + 1) * DQ, :],
                       preferred_element_type=_F32)
        acc = part if acc is None else acc + part
    out_ref[...] = acc + bc_ref[...]


def _tc3(S2, g2, deg2, b2r, Wc, bc):
    return pl.pallas_call(
        _tc3_body,
        grid=(NP // R,),
        in_specs=[
            pl.BlockSpec((2, R, DQ), lambda i: (0, i, 0)),
            pl.BlockSpec((2, R, DQ), lambda i: (0, i, 0)),
            pl.BlockSpec((NC, R), lambda i: (0, i)),
            pl.BlockSpec((1, OUT_DIM), lambda i: (0, 0)),
            pl.BlockSpec((OUT_DIM, PROJ_DIM), lambda i: (0, 0)),
            pl.BlockSpec((1, PROJ_DIM), lambda i: (0, 0)),
        ],
        out_specs=pl.BlockSpec((R, PROJ_DIM), lambda i: (i, 0)),
        out_shape=jax.ShapeDtypeStruct((NP, PROJ_DIM), _F32),
    )(S2, g2, deg2, b2r, Wc, bc)


def kernel(x, edge_index, W1, b1, W2, b2, Wf1, bf1, Wf2, bf2):
    src = edge_index[0]
    dst = edge_index[1]
    pad = jnp.full((EP - E,), PAD_ROW, dtype=jnp.int32)
    src1 = jnp.concatenate([src, pad]).reshape(EP // CHUNK, CHUNK)
    dst2 = jnp.concatenate([dst, pad]).reshape(EP // CHUNK, CHUNK)

    x_pad = jnp.zeros((NP, IN_DIM), _F32).at[:N].set(x)
    b1r = b1.reshape(1, MID_DIM)
    b2r = b2.reshape(1, OUT_DIM)
    bf1r = bf1.reshape(1, 256)
    bf2r = bf2.reshape(1, PROJ_DIM)
    zrows = jnp.zeros((CHUNK, DQ), _F32)
    zrow1 = jnp.zeros((1, ROWS_PER_SUB), _F32)

    deg2 = _deg_kernel(dst2, zrow1).reshape(NC, NP)

    g1, Wc, bc = _tc1(x_pad, W1, deg2, Wf1, Wf2, bf1r, bf2r)

    S1 = _scatter_kernel(g1.reshape(4 * NP, DQ), src1, dst2, zrows,
                         nphases=2).reshape(4, NP, DQ)

    g2 = _tc2(S1, g1, deg2, b1r, W2)

    S2 = _scatter_kernel(g2.reshape(2 * NP, DQ), src1, dst2, zrows,
                         nphases=1).reshape(2, NP, DQ)

    out = _tc3(S2, g2, deg2, b2r, Wc, bc)
    return out[:N]
